# baseline (speedup 1.0000x reference)
import numpy as np

import jax
import jax.numpy as jnp
from jax.experimental import pallas as pl
from jax.experimental.pallas import tpu as pltpu
from jax.experimental.shard_map import shard_map
from jax.sharding import Mesh, PartitionSpec as P

BM = 1024
BN = 2048
BK = 1024


def _masked_linear_kernel(d_ref, w_ref, m_ref, b_ref, o_ref):
    k = pl.program_id(2)
    w = w_ref[...] * m_ref[...]
    d = d_ref[...].astype(jnp.bfloat16)
    prod = jax.lax.dot_general(
        d, w,
        dimension_numbers=(((1,), (0,)), ((), ())),
        preferred_element_type=jnp.float32,
    )

    @pl.when(k == 0)
    def _init():
        o_ref[...] = prod + b_ref[...]

    @pl.when(k > 0)
    def _acc():
        o_ref[...] += prod


def _masked_linear(data, wt16, mt16, b2):
    M, K = data.shape
    N = wt16.shape[1]
    bm, bn, bk = min(BM, M), min(BN, N), min(BK, K)
    grid = (N // bn, M // bm, K // bk)
    return pl.pallas_call(
        _masked_linear_kernel,
        grid=grid,
        in_specs=[
            pl.BlockSpec((bm, bk), lambda j, i, k: (i, k)),
            pl.BlockSpec((bk, bn), lambda j, i, k: (k, j)),
            pl.BlockSpec((bk, bn), lambda j, i, k: (k, j)),
            pl.BlockSpec((1, bn), lambda j, i, k: (0, j)),
        ],
        out_specs=pl.BlockSpec((bm, bn), lambda j, i, k: (i, j)),
        out_shape=jax.ShapeDtypeStruct((M, N), jnp.float32),
        compiler_params=pltpu.CompilerParams(
            dimension_semantics=("parallel", "parallel", "arbitrary"),
        ),
    )(data, wt16, mt16, b2)


def kernel(data, w_mask, weight, bias):
    N = weight.shape[0]
    wt16 = weight.T.astype(jnp.bfloat16)
    mt16 = w_mask.T.astype(jnp.bfloat16)
    b2 = bias.reshape(1, N)

    devs = jax.devices()
    if len(devs) < 2:
        return _masked_linear(data, wt16, mt16, b2)

    mesh = Mesh(np.array(devs[:2]), ("x",))
    f = shard_map(
        _masked_linear,
        mesh=mesh,
        in_specs=(P("x", None), P(), P(), P()),
        out_specs=P("x", None),
        check_rep=False,
    )
    return f(data, wt16, mt16, b2)


# speedup vs baseline: 1.7577x; 1.7577x over previous
import jax
import jax.numpy as jnp
from jax.experimental import pallas as pl
from jax.experimental.pallas import tpu as pltpu

BM = 1024
BN = 2048
BK = 1024


def _masked_linear_kernel(d_ref, w_ref, m_ref, b_ref, o_ref):
    k = pl.program_id(2)
    w = w_ref[...] * m_ref[...]
    d = d_ref[...].astype(jnp.bfloat16)
    prod = jax.lax.dot_general(
        d, w,
        dimension_numbers=(((1,), (0,)), ((), ())),
        preferred_element_type=jnp.float32,
    )

    @pl.when(k == 0)
    def _init():
        o_ref[...] = prod + b_ref[...]

    @pl.when(k > 0)
    def _acc():
        o_ref[...] += prod


def kernel(data, w_mask, weight, bias):
    M, K = data.shape
    N = weight.shape[0]
    bm, bn, bk = min(BM, M), min(BN, N), min(BK, K)

    wt16 = weight.T.astype(jnp.bfloat16)
    mt16 = w_mask.T.astype(jnp.bfloat16)
    b2 = bias.reshape(1, N)

    grid = (N // bn, M // bm, K // bk)
    return pl.pallas_call(
        _masked_linear_kernel,
        grid=grid,
        in_specs=[
            pl.BlockSpec((bm, bk), lambda j, i, k: (i, k)),
            pl.BlockSpec((bk, bn), lambda j, i, k: (k, j)),
            pl.BlockSpec((bk, bn), lambda j, i, k: (k, j)),
            pl.BlockSpec((1, bn), lambda j, i, k: (0, j)),
        ],
        out_specs=pl.BlockSpec((bm, bn), lambda j, i, k: (i, j)),
        out_shape=jax.ShapeDtypeStruct((M, N), jnp.float32),
        compiler_params=pltpu.CompilerParams(
            dimension_semantics=("parallel", "parallel", "arbitrary"),
        ),
    )(data, wt16, mt16, b2)
